# baseline (speedup 1.0000x reference)
import functools

import jax
import jax.numpy as jnp
from jax import lax
from jax.experimental import pallas as pl
from jax.experimental.pallas import tpu as pltpu
from jax.experimental.pallas import tpu_sc as plsc

HIDDEN = 64

NC = 2
NS = 16
NW = NC * NS
CHUNK = 125
NCH = 250
ROWS_W = NCH * CHUNK
N_TOTAL = NW * ROWS_W
NBUF = 10
LOOKAHEAD = 5

HB = 4096
WIN = 2 * HB
NWIN = N_TOTAL // WIN
TAIL0 = NWIN * WIN
TAILW = N_TOTAL - TAIL0
TAILH = TAILW // 2


def _cdot(a_t, w, bias):
    return (
        jax.lax.dot_general(
            a_t, w, (((0,), (0,)), ((), ())),
            preferred_element_type=jnp.float32,
        )
        + bias
    )


def _transform_body(l_ref, r_ref, tail_ref, w_ref, b_ref, out_ref):
    i = pl.program_id(0)
    w = w_ref[...]
    bias = b_ref[...]

    @pl.when(i < NWIN)
    def _main():
        out_ref[:, :HIDDEN] = _cdot(l_ref[...], w, bias)
        out_ref[:, HIDDEN:] = _cdot(r_ref[...], w, bias)

    @pl.when(i == NWIN)
    def _tail():
        c = (
            jnp.dot(tail_ref[...], w, preferred_element_type=jnp.float32)
            + bias
        )
        out_ref[0:TAILH, :HIDDEN] = c[0:TAILH]
        out_ref[0:TAILH, HIDDEN:] = c[TAILH:TAILW]


def _transform_table(node_emb, W, b):
    n = node_emb.shape[0]
    n2 = n // 2
    last = n // HB - 1
    tail = lax.slice(node_emb, (TAIL0, 0), (n, HIDDEN))
    return pl.pallas_call(
        _transform_body,
        grid=(NWIN + 1,),
        in_specs=[
            pl.BlockSpec(
                (HIDDEN, HB), lambda i: (0, jnp.minimum(2 * i, last))),
            pl.BlockSpec(
                (HIDDEN, HB), lambda i: (0, jnp.minimum(2 * i + 1, last))),
            pl.BlockSpec((TAILW, HIDDEN), lambda i: (0, 0)),
            pl.BlockSpec((HIDDEN, HIDDEN), lambda i: (0, 0)),
            pl.BlockSpec((1, HIDDEN), lambda i: (0, 0)),
        ],
        out_specs=pl.BlockSpec((HB, 2 * HIDDEN), lambda i: (i, 0)),
        out_shape=jax.ShapeDtypeStruct((n2, 2 * HIDDEN), jnp.float32),
    )(node_emb.T, node_emb.T, tail, W, b.reshape(1, HIDDEN))


def _gather_body(table_hbm, idx_hbm, out_hbm, idx_v, rows_v, gsem, osem):
    wid = lax.axis_index("s") * NC + lax.axis_index("c")
    pltpu.sync_copy(idx_hbm.at[pl.ds(wid * NCH, NCH)], idx_v)
    row0 = wid * ROWS_W

    def gather(c, b):
        return pltpu.make_async_copy(
            table_hbm.at[idx_v.at[c]], rows_v.at[b], gsem.at[b])

    def outcopy(c, b):
        return pltpu.make_async_copy(
            rows_v.at[b], out_hbm.at[pl.ds(row0 + c * CHUNK, CHUNK)],
            osem.at[b])

    for b in range(LOOKAHEAD):
        gather(b, b).start()

    def outer(o, carry):
        for b in range(NBUF):
            c = o * NBUF + b
            gather(c, b).wait()
            outcopy(c, b).start()
            g = c + LOOKAHEAD
            bg = (b + LOOKAHEAD) % NBUF

            @pl.when(g < NCH)
            def _start_ahead(g=g, bg=bg):
                @pl.when(g >= NBUF)
                def _reuse(g=g, bg=bg):
                    outcopy(g - NBUF, bg).wait()

                gather(g, bg).start()
        return carry

    lax.fori_loop(0, NCH // NBUF, outer, 0)

    for b in range(NBUF):
        outcopy(NCH - NBUF + b, b).wait()


@functools.lru_cache(maxsize=1)
def _make_gather_kernel():
    return pl.kernel(
        _gather_body,
        mesh=plsc.VectorSubcoreMesh(core_axis_name="c", subcore_axis_name="s"),
        compiler_params=pltpu.CompilerParams(use_tc_tiling_on_sc=False),
        out_type=jax.ShapeDtypeStruct((N_TOTAL, HIDDEN), jnp.float32),
        scratch_types=[
            pltpu.VMEM((NCH, CHUNK), jnp.int32),
            pltpu.VMEM((NBUF, CHUNK, HIDDEN), jnp.float32),
            pltpu.SemaphoreType.DMA((NBUF,)),
            pltpu.SemaphoreType.DMA((NBUF,)),
        ],
    )


def _emit_body(x_ref, out_ref):
    i = pl.program_id(0)

    def halves_t(xv):
        lt = xv[:, :HIDDEN].T
        rt = xv[:, HIDDEN:].T
        return lt, rt

    @pl.when(i < NWIN)
    def _main():
        lt, rt = halves_t(x_ref[...])
        out_ref[:, :HB] = lt
        out_ref[:, HB:] = rt

    @pl.when(i == NWIN)
    def _tail():
        lt, rt = halves_t(x_ref[...])
        out_ref[:, 0:TAILH] = lt[:, 0:TAILH]
        out_ref[:, TAILH:TAILW] = rt[:, 0:TAILH]


def _emit_transposed(h):
    return pl.pallas_call(
        _emit_body,
        grid=(NWIN + 1,),
        in_specs=[pl.BlockSpec((HB, 2 * HIDDEN), lambda i: (i, 0))],
        out_specs=pl.BlockSpec((HIDDEN, WIN), lambda i: (0, i)),
        out_shape=jax.ShapeDtypeStruct((HIDDEN, N_TOTAL), jnp.float32),
    )(h.reshape(N_TOTAL // 2, 2 * HIDDEN))


def _slot_permute(v):
    a = v[:TAIL0].reshape(NWIN, 2, HB).transpose(0, 2, 1).reshape(-1)
    t = v[TAIL0:].reshape(2, TAILH).transpose(1, 0).reshape(-1)
    return jnp.concatenate([a, t])


def kernel(x, node_emb, W, b):
    table2 = _transform_table(node_emb, W, b)
    xi = x.astype(jnp.int32)
    t = xi % WIN
    j_main = (xi - t) + 2 * (t % HB) + (t // HB)
    tt = xi - TAIL0
    j_tail = TAIL0 + 2 * (tt % TAILH) + (tt // TAILH)
    idx = jnp.where(xi < TAIL0, j_main, j_tail)
    idx2d = _slot_permute(idx).reshape(NW * NCH, CHUNK)
    h = _make_gather_kernel()(table2.reshape(N_TOTAL, HIDDEN), idx2d)
    return _emit_transposed(h).T


# speedup vs baseline: 1.0488x; 1.0488x over previous
import functools

import jax
import jax.numpy as jnp
from jax import lax
from jax.experimental import pallas as pl
from jax.experimental.pallas import tpu as pltpu
from jax.experimental.pallas import tpu_sc as plsc

HIDDEN = 64

NC = 2
NS = 16
NW = NC * NS
CHUNK = 125
NCH = 250
ROWS_W = NCH * CHUNK
N_TOTAL = NW * ROWS_W
NBUF = 10
LOOKAHEAD = 5

HB = 4096
WIN = 2 * HB
NWIN = N_TOTAL // WIN
TAIL0 = NWIN * WIN
TAILW = N_TOTAL - TAIL0
TAILH = TAILW // 2


def _cdot(a_t, w, bias):
    return (
        jax.lax.dot_general(
            a_t, w, (((0,), (0,)), ((), ())),
            preferred_element_type=jnp.float32,
        )
        + bias
    )


def _transform_body(l_ref, r_ref, tail_ref, w_ref, b_ref, out_ref):
    i = pl.program_id(0)
    w = w_ref[...]
    bias = b_ref[...]

    @pl.when(i < NWIN)
    def _main():
        out_ref[:, :HIDDEN] = _cdot(l_ref[...], w, bias)
        out_ref[:, HIDDEN:] = _cdot(r_ref[...], w, bias)

    @pl.when(i == NWIN)
    def _tail():
        c = (
            jnp.dot(tail_ref[...], w, preferred_element_type=jnp.float32)
            + bias
        )
        out_ref[0:TAILH, :HIDDEN] = c[0:TAILH]
        out_ref[0:TAILH, HIDDEN:] = c[TAILH:TAILW]


def _transform_table(node_emb, W, b):
    n = node_emb.shape[0]
    n2 = n // 2
    last = n // HB - 1
    tail = lax.slice(node_emb, (TAIL0, 0), (n, HIDDEN))
    return pl.pallas_call(
        _transform_body,
        grid=(NWIN + 1,),
        in_specs=[
            pl.BlockSpec(
                (HIDDEN, HB), lambda i: (0, jnp.minimum(2 * i, last))),
            pl.BlockSpec(
                (HIDDEN, HB), lambda i: (0, jnp.minimum(2 * i + 1, last))),
            pl.BlockSpec((TAILW, HIDDEN), lambda i: (0, 0)),
            pl.BlockSpec((HIDDEN, HIDDEN), lambda i: (0, 0)),
            pl.BlockSpec((1, HIDDEN), lambda i: (0, 0)),
        ],
        out_specs=pl.BlockSpec((HB, 2 * HIDDEN), lambda i: (i, 0)),
        out_shape=jax.ShapeDtypeStruct((n2, 2 * HIDDEN), jnp.float32),
    )(node_emb.T, node_emb.T, tail, W, b.reshape(1, HIDDEN))


def _gather_body(table_hbm, idx_hbm, out_hbm, idx_v, rows_v, gsem, osem):
    wid = lax.axis_index("s") * NC + lax.axis_index("c")
    pltpu.sync_copy(idx_hbm.at[pl.ds(wid * NCH, NCH)], idx_v)
    row0 = wid * ROWS_W

    def gather(c, b):
        return pltpu.make_async_copy(
            table_hbm.at[idx_v.at[c]], rows_v.at[b], gsem.at[b])

    def outcopy(c, b):
        return pltpu.make_async_copy(
            rows_v.at[b], out_hbm.at[pl.ds(row0 + c * CHUNK, CHUNK)],
            osem.at[b])

    for b in range(LOOKAHEAD):
        gather(b, b).start()

    def outer(o, carry):
        for b in range(NBUF):
            c = o * NBUF + b
            gather(c, b).wait()
            outcopy(c, b).start()
            g = c + LOOKAHEAD
            bg = (b + LOOKAHEAD) % NBUF

            @pl.when(g < NCH)
            def _start_ahead(g=g, bg=bg):
                @pl.when(g >= NBUF)
                def _reuse(g=g, bg=bg):
                    outcopy(g - NBUF, bg).wait()

                gather(g, bg).start()
        return carry

    lax.fori_loop(0, NCH // NBUF, outer, 0)

    for b in range(NBUF):
        outcopy(NCH - NBUF + b, b).wait()


@functools.lru_cache(maxsize=1)
def _make_gather_kernel():
    return pl.kernel(
        _gather_body,
        mesh=plsc.VectorSubcoreMesh(core_axis_name="c", subcore_axis_name="s"),
        compiler_params=pltpu.CompilerParams(use_tc_tiling_on_sc=False),
        out_type=jax.ShapeDtypeStruct((N_TOTAL, HIDDEN), jnp.float32),
        scratch_types=[
            pltpu.VMEM((NCH, CHUNK), jnp.int32),
            pltpu.VMEM((NBUF, CHUNK, HIDDEN), jnp.float32),
            pltpu.SemaphoreType.DMA((NBUF,)),
            pltpu.SemaphoreType.DMA((NBUF,)),
        ],
    )


def _emit_body(x_ref, out_ref):
    i = pl.program_id(0)

    def halves_t(xv):
        eye = jnp.eye(HIDDEN, dtype=jnp.float32)
        lt = jax.lax.dot_general(
            eye, xv[:, :HIDDEN],
            (((1,), (1,)), ((), ())), preferred_element_type=jnp.float32)
        rt = jax.lax.dot_general(
            eye, xv[:, HIDDEN:],
            (((1,), (1,)), ((), ())), preferred_element_type=jnp.float32)
        return lt, rt

    @pl.when(i < NWIN)
    def _main():
        lt, rt = halves_t(x_ref[...])
        out_ref[:, :HB] = lt
        out_ref[:, HB:] = rt

    @pl.when(i == NWIN)
    def _tail():
        lt, rt = halves_t(x_ref[...])
        out_ref[:, 0:TAILH] = lt[:, 0:TAILH]
        out_ref[:, TAILH:TAILW] = rt[:, 0:TAILH]


def _emit_transposed(h):
    return pl.pallas_call(
        _emit_body,
        grid=(NWIN + 1,),
        in_specs=[pl.BlockSpec((HB, 2 * HIDDEN), lambda i: (i, 0))],
        out_specs=pl.BlockSpec((HIDDEN, WIN), lambda i: (0, i)),
        out_shape=jax.ShapeDtypeStruct((HIDDEN, N_TOTAL), jnp.float32),
    )(h.reshape(N_TOTAL // 2, 2 * HIDDEN))


def _slot_permute(v):
    a = v[:TAIL0].reshape(NWIN, 2, HB).transpose(0, 2, 1).reshape(-1)
    t = v[TAIL0:].reshape(2, TAILH).transpose(1, 0).reshape(-1)
    return jnp.concatenate([a, t])


def kernel(x, node_emb, W, b):
    table2 = _transform_table(node_emb, W, b)
    xi = x.astype(jnp.int32)
    t = xi % WIN
    j_main = (xi - t) + 2 * (t % HB) + (t // HB)
    tt = xi - TAIL0
    j_tail = TAIL0 + 2 * (tt % TAILH) + (tt // TAILH)
    idx = jnp.where(xi < TAIL0, j_main, j_tail)
    idx2d = _slot_permute(idx).reshape(NW * NCH, CHUNK)
    h = _make_gather_kernel()(table2.reshape(N_TOTAL, HIDDEN), idx2d)
    return _emit_transposed(h).T
